# baseline (speedup 1.0000x reference)
import functools

import jax
import jax.numpy as jnp
from jax.experimental import pallas as pl
from jax.experimental.pallas import tpu as pltpu

LEAK = 0.1
E_BLK = 8000


def _lrelu(h):
    return jnp.where(h > 0, h, LEAK * h)


def _bn(v, g, b):
    mu = v.mean(axis=0)
    var = v.var(axis=0)
    return g * (v - mu) / jnp.sqrt(var + 1e-5) + b


def _edge_mlp_body(h_ref, w0, b0, w1, b1, w2, b2, w3, b3, w4, b4, o_ref):
    h = h_ref[...]
    h = _lrelu(jnp.dot(h, w0[...].T, preferred_element_type=jnp.float32) + b0[...])
    h = _lrelu(jnp.dot(h, w1[...].T, preferred_element_type=jnp.float32) + b1[...])
    h = _lrelu(jnp.dot(h, w2[...].T, preferred_element_type=jnp.float32) + b2[...])
    h = _lrelu(jnp.dot(h, w3[...].T, preferred_element_type=jnp.float32) + b3[...])
    o_ref[...] = jnp.dot(h, w4[...].T, preferred_element_type=jnp.float32) + b4[...]


def _edge_mlp(h, params):
    E = h.shape[0]
    grid = (E // E_BLK,)
    win = [pl.BlockSpec((E_BLK, h.shape[1]), lambda i: (i, 0))]
    for i in range(5):
        w = params[f'ep_w{i}']
        win.append(pl.BlockSpec(w.shape, lambda i: (0, 0)))
        win.append(pl.BlockSpec(params[f'ep_b{i}'].shape, lambda i: (0,)))
    args = [h]
    for i in range(5):
        args += [params[f'ep_w{i}'], params[f'ep_b{i}']]
    return pl.pallas_call(
        _edge_mlp_body,
        grid=grid,
        in_specs=win,
        out_specs=pl.BlockSpec((E_BLK, 2), lambda i: (i, 0)),
        out_shape=jax.ShapeDtypeStruct((E, 2), jnp.float32),
    )(*args)


def _mlp3(h, p, pre):
    h = _lrelu(h @ p[pre + '_w0'].T + p[pre + '_b0'])
    h = _lrelu(h @ p[pre + '_w1'].T + p[pre + '_b1'])
    return h @ p[pre + '_w2'].T + p[pre + '_b2']


def _edge_conv(x, src, dst, p, pre, n):
    xi = x[dst]
    xj = x[src]
    m = _mlp3(jnp.concatenate([xi, xj - xi], axis=1), p, pre)
    out = jax.ops.segment_max(m, dst, num_segments=n)
    return jnp.where(jnp.isneginf(out), 0.0, out)


def kernel(x, edge_index, e, xbatch, params):
    src = edge_index[0]
    dst = edge_index[1]
    n = x.shape[0]
    x = _bn(x, params['bn_node_g'], params['bn_node_b'])
    e = _bn(e, params['bn_edge_g'], params['bn_edge_b'])
    x = _edge_conv(x, src, dst, params, 'nn0', n)
    x = _edge_conv(x, src, dst, params, 'nn1', n)
    x = _edge_conv(x, src, dst, params, 'nn2', n)
    h = jnp.concatenate([x[src], x[dst], e], axis=1)
    return _edge_mlp(h, params)


# speedup vs baseline: 1.6989x; 1.6989x over previous
import functools

import jax
import jax.numpy as jnp
from jax import lax
from jax.experimental import pallas as pl
from jax.experimental.pallas import tpu as pltpu
from jax.experimental.pallas import tpu_sc as plsc

LEAK = 0.1
E_BLK = 8000

_SC_MESH = dict(core_axis_name="c", subcore_axis_name="s")
NW = 32


@functools.lru_cache(maxsize=None)
def _make_pair_gather(N, F, E, W=1000):
    per_w = E // NW
    assert per_w % W == 0 and W % 8 == 0
    nwin = per_w // W
    mesh = plsc.VectorSubcoreMesh(**_SC_MESH)

    @functools.partial(
        pl.kernel,
        mesh=mesh,
        compiler_params=pltpu.CompilerParams(use_tc_tiling_on_sc=False),
        out_type=(
            jax.ShapeDtypeStruct((E, F), jnp.float32),
            jax.ShapeDtypeStruct((E, F), jnp.float32),
        ),
        scratch_types=[
            pltpu.VMEM((W,), jnp.int32),
            pltpu.VMEM((W, F), jnp.float32),
            pltpu.SemaphoreType.DMA,
        ],
    )
    def k(x_hbm, src_hbm, dst_hbm, osrc_hbm, odst_hbm, idx_v, rows_v, sem):
        wid = lax.axis_index("s") * 2 + lax.axis_index("c")
        base = wid * per_w

        def body(i, _):
            off = base + i * W
            pltpu.sync_copy(src_hbm.at[pl.ds(off, W)], idx_v)
            pltpu.async_copy(x_hbm.at[idx_v], rows_v, sem).wait()
            pltpu.sync_copy(rows_v, osrc_hbm.at[pl.ds(off, W)])
            pltpu.sync_copy(dst_hbm.at[pl.ds(off, W)], idx_v)
            pltpu.async_copy(x_hbm.at[idx_v], rows_v, sem).wait()
            pltpu.sync_copy(rows_v, odst_hbm.at[pl.ds(off, W)])
            return ()

        lax.fori_loop(0, nwin, body, (), unroll=False)

    return k


def _pair_gather(x, src, dst):
    N, F = x.shape
    E = src.shape[0]
    return _make_pair_gather(N, F, E)(x, src, dst)


def _lrelu(h):
    return jnp.where(h > 0, h, LEAK * h)


def _bn(v, g, b):
    mu = v.mean(axis=0)
    var = v.var(axis=0)
    return g * (v - mu) / jnp.sqrt(var + 1e-5) + b


def _edge_mlp_body(h_ref, w0, b0, w1, b1, w2, b2, w3, b3, w4, b4, o_ref):
    h = h_ref[...]
    h = _lrelu(jnp.dot(h, w0[...].T, preferred_element_type=jnp.float32) + b0[...])
    h = _lrelu(jnp.dot(h, w1[...].T, preferred_element_type=jnp.float32) + b1[...])
    h = _lrelu(jnp.dot(h, w2[...].T, preferred_element_type=jnp.float32) + b2[...])
    h = _lrelu(jnp.dot(h, w3[...].T, preferred_element_type=jnp.float32) + b3[...])
    o_ref[...] = jnp.dot(h, w4[...].T, preferred_element_type=jnp.float32) + b4[...]


def _edge_mlp(h, params):
    E = h.shape[0]
    grid = (E // E_BLK,)
    win = [pl.BlockSpec((E_BLK, h.shape[1]), lambda i: (i, 0))]
    for i in range(5):
        w = params[f'ep_w{i}']
        win.append(pl.BlockSpec(w.shape, lambda i: (0, 0)))
        win.append(pl.BlockSpec(params[f'ep_b{i}'].shape, lambda i: (0,)))
    args = [h]
    for i in range(5):
        args += [params[f'ep_w{i}'], params[f'ep_b{i}']]
    return pl.pallas_call(
        _edge_mlp_body,
        grid=grid,
        in_specs=win,
        out_specs=pl.BlockSpec((E_BLK, 2), lambda i: (i, 0)),
        out_shape=jax.ShapeDtypeStruct((E, 2), jnp.float32),
    )(*args)


def _mlp3(h, p, pre):
    h = _lrelu(h @ p[pre + '_w0'].T + p[pre + '_b0'])
    h = _lrelu(h @ p[pre + '_w1'].T + p[pre + '_b1'])
    return h @ p[pre + '_w2'].T + p[pre + '_b2']


def _edge_conv(x, src, dst, p, pre, n):
    xj, xi = _pair_gather(x, src, dst)
    m = _mlp3(jnp.concatenate([xi, xj - xi], axis=1), p, pre)
    out = jax.ops.segment_max(m, dst, num_segments=n)
    return jnp.where(jnp.isneginf(out), 0.0, out)


def kernel(x, edge_index, e, xbatch, params):
    src = edge_index[0]
    dst = edge_index[1]
    n = x.shape[0]
    x = _bn(x, params['bn_node_g'], params['bn_node_b'])
    e = _bn(e, params['bn_edge_g'], params['bn_edge_b'])
    x = _edge_conv(x, src, dst, params, 'nn0', n)
    x = _edge_conv(x, src, dst, params, 'nn1', n)
    x = _edge_conv(x, src, dst, params, 'nn2', n)
    hs, hd = _pair_gather(x, src, dst)
    h = jnp.concatenate([hs, hd, e], axis=1)
    return _edge_mlp(h, params)
